# baseline (speedup 1.0000x reference)
import functools

import jax
import jax.numpy as jnp
from jax import lax
from jax.experimental import pallas as pl
from jax.experimental.pallas import tpu as pltpu
from jax.experimental.pallas import tpu_sc as plsc

B, L = 16384, 200
VOCAB, DIM = 10, 3
NC, NS = 2, 16
NW = NC * NS
RPW = B // NW
CH = 64
NCHUNK = RPW // CH
LT = 192
TVR = CH * 8 // 128

_mesh = plsc.VectorSubcoreMesh(core_axis_name="c", subcore_axis_name="s")


@functools.partial(
    pl.kernel,
    mesh=_mesh,
    out_type=jax.ShapeDtypeStruct((B * DIM,), jnp.float32),
    scratch_types=[
        pltpu.VMEM((2, 2, CH, L), jnp.int32),
        pltpu.VMEM((2 * CH * 16,), jnp.float32),
        pltpu.VMEM((2 * VOCAB * DIM, 16), jnp.float32),
        pltpu.VMEM((CH * DIM,), jnp.float32),
        pltpu.SemaphoreType.DMA,
        pltpu.SemaphoreType.DMA,
        pltpu.SemaphoreType.DMA,
        pltpu.SemaphoreType.DMA,
    ],
    compiler_params=pltpu.CompilerParams(needs_layout_passes=False),
)
def _sc_merge(
    f1_hbm, f2_hbm, tb_hbm, out_hbm, fb, hist, tbv, ob,
    sem_s0f1, sem_s0f2, sem_s1f1, sem_s1f2,
):
    sem_in = ((sem_s0f1, sem_s0f2), (sem_s1f1, sem_s1f2))
    f_hbm = (f1_hbm, f2_hbm)
    wid = lax.axis_index("s") * NC + lax.axis_index("c")
    pltpu.sync_copy(tb_hbm, tbv)
    iota = lax.iota(jnp.int32, 16)
    ones = jnp.ones((16,), jnp.float32)
    zeros = jnp.zeros((16,), jnp.float32)
    def start_in(c, slot):
        row0 = pl.multiple_of(wid * RPW + c * CH, CH)
        for f in (0, 1):
            pltpu.async_copy(
                f_hbm[f].at[pl.ds(row0, CH)], fb.at[slot, f], sem_in[slot][f]
            )

    def wait_in(slot):
        for f in (0, 1):
            pltpu.make_async_copy(
                f_hbm[f].at[pl.ds(0, CH)], fb.at[slot, f], sem_in[slot][f]
            ).wait()

    start_in(0, 0)
    start_in(1, 1)

    def chunk2(c2, _):
        for sl in (0, 1):
            c = c2 * 2 + sl
            wait_in(sl)

            @plsc.parallel_loop(0, 2 * CH, unroll=8)
            def _zero(i):
                hist[pl.ds(i * 16, 16)] = zeros

            @plsc.parallel_loop(0, CH)
            def _p1row(r, sl=sl):
                bvec = jnp.full((16,), 0, jnp.int32) + r * 32
                for blk in range(0, 12, 4):
                    idx = []
                    for k in range(blk, blk + 4):
                        v1 = fb[sl, 0, r, pl.ds(k * 16, 16)]
                        v2 = fb[sl, 1, r, pl.ds(k * 16, 16)]
                        idx.append((bvec + v1, bvec + 16 + v2))
                    for i1, i2 in idx:
                        plsc.addupdate_scatter(hist, [i1], ones)
                        plsc.addupdate_scatter(hist, [i2], ones)

            @pl.when(c2 < (NCHUNK // 2 - 1))
            def _():
                start_in(c + 2, sl)

            @plsc.parallel_loop(0, CH // 16)
            def _p2(g):
                rows = g * 16 + iota
                rbins = rows * 32
                acc = [zeros, zeros, zeros]
                for v in range(VOCAB):
                    c1 = plsc.load_gather(hist, [rbins + v])
                    c2v = plsc.load_gather(hist, [rbins + (16 + v)])
                    for d in range(DIM):
                        acc[d] = acc[d] + c1 * tbv[v * DIM + d] + c2v * tbv[(VOCAB + v) * DIM + d]
                rows3 = rows * 3
                for d in range(DIM):
                    plsc.store_scatter(ob, [rows3 + d], acc[d])

            pltpu.sync_copy(
                ob, out_hbm.at[pl.ds((wid * RPW + c * CH) * DIM, CH * DIM)]
            )
        return 0

    lax.fori_loop(0, NCHUNK // 2, chunk2, 0)


TROWS = 2048


def _tc_tail_body(f1_ref, f2_ref, t1_ref, t2_ref, o_ref):
    f1 = f1_ref[:, 192:200]
    f2 = f2_ref[:, 192:200]
    h1 = jnp.stack(
        [jnp.sum((f1 == v).astype(jnp.float32), axis=1) for v in range(VOCAB)],
        axis=1,
    )
    h2 = jnp.stack(
        [jnp.sum((f2 == v).astype(jnp.float32), axis=1) for v in range(VOCAB)],
        axis=1,
    )
    acc = jnp.dot(h1, t1_ref[...], preferred_element_type=jnp.float32)
    acc += jnp.dot(h2, t2_ref[...], preferred_element_type=jnp.float32)
    o_ref[...] = acc * jnp.float32(1.0 / L)


def _tc_tail(feature_1, feature_2, table_1, table_2):
    return pl.pallas_call(
        _tc_tail_body,
        grid=(B // TROWS,),
        in_specs=[
            pl.BlockSpec((TROWS, L), lambda i: (i, 0)),
            pl.BlockSpec((TROWS, L), lambda i: (i, 0)),
            pl.BlockSpec((VOCAB, DIM), lambda i: (0, 0)),
            pl.BlockSpec((VOCAB, DIM), lambda i: (0, 0)),
        ],
        out_specs=pl.BlockSpec((TROWS, DIM), lambda i: (i, 0)),
        out_shape=jax.ShapeDtypeStruct((B, DIM), jnp.float32),
    )(feature_1, feature_2, table_1, table_2)


def kernel(feature_1, feature_2, table_1, table_2):
    tb = jnp.concatenate([table_1.reshape(-1), table_2.reshape(-1)])
    tb = jnp.broadcast_to((tb * jnp.float32(1.0 / L))[:, None], (2 * VOCAB * DIM, 16))
    main = _sc_merge(feature_1, feature_2, tb).reshape(B, DIM)
    tail = _tc_tail(feature_1, feature_2, table_1, table_2)
    return main + tail


# speedup vs baseline: 1.0850x; 1.0850x over previous
import functools

import jax
import jax.numpy as jnp
from jax import lax
from jax.experimental import pallas as pl
from jax.experimental.pallas import tpu as pltpu
from jax.experimental.pallas import tpu_sc as plsc

B, L = 16384, 200
VOCAB, DIM = 10, 3
NC, NS = 2, 16
NW = NC * NS
RPW = B // NW
CH = 64
NCHUNK = RPW // CH
LT = 192
TVR = CH * 8 // 128

_mesh = plsc.VectorSubcoreMesh(core_axis_name="c", subcore_axis_name="s")


@functools.partial(
    pl.kernel,
    mesh=_mesh,
    out_type=jax.ShapeDtypeStruct((B * DIM,), jnp.float32),
    scratch_types=[
        pltpu.VMEM((2, 2, CH, L), jnp.int32),
        pltpu.VMEM((2 * CH * 16,), jnp.float32),
        pltpu.VMEM((2 * VOCAB * DIM, 16), jnp.float32),
        pltpu.VMEM((CH * DIM,), jnp.float32),
        pltpu.SemaphoreType.DMA,
        pltpu.SemaphoreType.DMA,
        pltpu.SemaphoreType.DMA,
        pltpu.SemaphoreType.DMA,
    ],
    compiler_params=pltpu.CompilerParams(needs_layout_passes=False),
)
def _sc_merge(
    f1_hbm, f2_hbm, tb_hbm, out_hbm, fb, hist, tbv, ob,
    sem_s0f1, sem_s0f2, sem_s1f1, sem_s1f2,
):
    sem_in = ((sem_s0f1, sem_s0f2), (sem_s1f1, sem_s1f2))
    f_hbm = (f1_hbm, f2_hbm)
    wid = lax.axis_index("s") * NC + lax.axis_index("c")
    pltpu.sync_copy(tb_hbm, tbv)
    iota = lax.iota(jnp.int32, 16)
    ones = jnp.ones((16,), jnp.float32)
    zeros = jnp.zeros((16,), jnp.float32)
    def start_in(c, slot):
        row0 = pl.multiple_of(wid * RPW + c * CH, CH)
        for f in (0, 1):
            pltpu.async_copy(
                f_hbm[f].at[pl.ds(row0, CH)], fb.at[slot, f], sem_in[slot][f]
            )

    def wait_in(slot):
        for f in (0, 1):
            pltpu.make_async_copy(
                f_hbm[f].at[pl.ds(0, CH)], fb.at[slot, f], sem_in[slot][f]
            ).wait()

    start_in(0, 0)
    start_in(1, 1)

    def chunk2(c2, _):
        for sl in (0, 1):
            c = c2 * 2 + sl
            wait_in(sl)

            @plsc.parallel_loop(0, 2 * CH, unroll=8)
            def _zero(i):
                hist[pl.ds(i * 16, 16)] = zeros

            @plsc.parallel_loop(0, CH)
            def _p1row(r, sl=sl):
                bvec = jnp.full((16,), 0, jnp.int32) + r * 32
                for blk in range(0, 12, 4):
                    idx = []
                    for k in range(blk, blk + 4):
                        v1 = fb[sl, 0, r, pl.ds(k * 16, 16)]
                        v2 = fb[sl, 1, r, pl.ds(k * 16, 16)]
                        idx.append((bvec + v1, bvec + 16 + v2))
                    for i1, i2 in idx:
                        plsc.addupdate_scatter(hist, [i1], ones)
                        plsc.addupdate_scatter(hist, [i2], ones)

            @pl.when(c2 < (NCHUNK // 2 - 1))
            def _():
                start_in(c + 2, sl)

            @plsc.parallel_loop(0, CH // 16)
            def _p2(g):
                rows = g * 16 + iota
                rbins = rows * 32
                acc = [zeros, zeros, zeros]
                for v in range(VOCAB):
                    c1 = plsc.load_gather(hist, [rbins + v])
                    c2v = plsc.load_gather(hist, [rbins + (16 + v)])
                    for d in range(DIM):
                        acc[d] = acc[d] + c1 * tbv[v * DIM + d] + c2v * tbv[(VOCAB + v) * DIM + d]
                rows3 = rows * 3
                for d in range(DIM):
                    plsc.store_scatter(ob, [rows3 + d], acc[d])

            pltpu.sync_copy(
                ob, out_hbm.at[pl.ds((wid * RPW + c * CH) * DIM, CH * DIM)]
            )
        return 0

    lax.fori_loop(0, NCHUNK // 2, chunk2, 0)


TROWS = 2048


def _tc_tail_body(f1_ref, f2_ref, t1_ref, t2_ref, o_ref):
    f1 = f1_ref[...]
    f2 = f2_ref[...]
    h1 = jnp.stack(
        [jnp.sum((f1 == v).astype(jnp.float32), axis=1) for v in range(VOCAB)],
        axis=1,
    )
    h2 = jnp.stack(
        [jnp.sum((f2 == v).astype(jnp.float32), axis=1) for v in range(VOCAB)],
        axis=1,
    )
    acc = jnp.dot(h1, t1_ref[...], preferred_element_type=jnp.float32)
    acc += jnp.dot(h2, t2_ref[...], preferred_element_type=jnp.float32)
    o_ref[...] = acc * jnp.float32(1.0 / L)


def _tc_tail(feature_1, feature_2, table_1, table_2):
    return pl.pallas_call(
        _tc_tail_body,
        grid=(B // TROWS,),
        in_specs=[
            pl.BlockSpec((TROWS, L - LT), lambda i: (i, 0)),
            pl.BlockSpec((TROWS, L - LT), lambda i: (i, 0)),
            pl.BlockSpec((VOCAB, DIM), lambda i: (0, 0)),
            pl.BlockSpec((VOCAB, DIM), lambda i: (0, 0)),
        ],
        out_specs=pl.BlockSpec((TROWS, DIM), lambda i: (i, 0)),
        out_shape=jax.ShapeDtypeStruct((B, DIM), jnp.float32),
    )(feature_1, feature_2, table_1, table_2)


def kernel(feature_1, feature_2, table_1, table_2):
    tb = jnp.concatenate([table_1.reshape(-1), table_2.reshape(-1)])
    tb = jnp.broadcast_to((tb * jnp.float32(1.0 / L))[:, None], (2 * VOCAB * DIM, 16))
    main = _sc_merge(feature_1, feature_2, tb).reshape(B, DIM)
    tail = _tc_tail(feature_1[:, LT:], feature_2[:, LT:], table_1, table_2)
    return main + tail
